# baseline (speedup 1.0000x reference)
import functools

import jax
import jax.numpy as jnp
from jax import lax
from jax.experimental import pallas as pl
from jax.experimental.pallas import tpu as pltpu
from jax.experimental.pallas import tpu_sc as plsc

_TOPK = 8
_NE = 64
_BLK = 2048
_N = 16384
_NSUB = 32
_ROWS = _N // _NSUB
_SLABS_PER_BLK = _BLK // _ROWS


def _logits_block(x_ref, w_ref, out_ref):
    x = x_ref[...]
    w = w_ref[...]
    logits = jax.lax.dot_general(
        w, x, (((1,), (1,)), ((), ())), preferred_element_type=jnp.float32
    )
    for j in range(_SLABS_PER_BLK):
        out_ref[j, :, :] = logits[:, j * _ROWS : (j + 1) * _ROWS]


def _tc_logits(x, w):
    grid = _N // _BLK
    return pl.pallas_call(
        _logits_block,
        grid=(grid,),
        in_specs=[
            pl.BlockSpec((_BLK, x.shape[1]), lambda i: (i, 0)),
            pl.BlockSpec((_NE, x.shape[1]), lambda i: (0, 0)),
        ],
        out_specs=pl.BlockSpec((_SLABS_PER_BLK, _NE, _ROWS), lambda i: (i, 0, 0)),
        out_shape=jax.ShapeDtypeStruct((_NSUB, _NE, _ROWS), jnp.float32),
    )(x, w)


_MESH = plsc.VectorSubcoreMesh(core_axis_name="c", subcore_axis_name="s")


@functools.partial(
    pl.kernel,
    mesh=_MESH,
    out_type=[
        jax.ShapeDtypeStruct((_NSUB, _TOPK, _ROWS), jnp.int32),
        jax.ShapeDtypeStruct((_NSUB, _TOPK, _ROWS), jnp.float32),
    ],
    scratch_types=[
        pltpu.VMEM((_NE, _ROWS), jnp.float32),
        pltpu.VMEM((_TOPK, _ROWS), jnp.int32),
        pltpu.VMEM((_TOPK, _ROWS), jnp.float32),
    ],
)
def _sc_topk(logits_hbm, idx_hbm, val_hbm, slab, oidx, oval):
    wid = lax.axis_index("s") * 2 + lax.axis_index("c")
    pltpu.sync_copy(logits_hbm.at[wid], slab)

    def group_body(g, carry):
        base = pl.multiple_of(g * 16, 16)
        sl = pl.ds(base, 16)
        m = slab[0, sl]
        for e in range(1, _NE):
            m = jnp.maximum(m, slab[e, sl])
        t = [jnp.full((16,), -jnp.inf, dtype=jnp.float32) for _ in range(_TOPK)]
        ti = [jnp.zeros((16,), dtype=jnp.int32) for _ in range(_TOPK)]
        s = jnp.zeros((16,), dtype=jnp.float32)
        for e in range(_NE):
            v = slab[e, sl]
            s = s + jnp.exp(v - m)
            ev = jnp.full((16,), e, dtype=jnp.int32)
            gt = [v > t[j] for j in range(_TOPK)]
            for j in range(_TOPK - 1, 0, -1):
                t[j] = jnp.where(gt[j], jnp.where(gt[j - 1], t[j - 1], v), t[j])
                ti[j] = jnp.where(gt[j], jnp.where(gt[j - 1], ti[j - 1], ev), ti[j])
            t[0] = jnp.where(gt[0], v, t[0])
            ti[0] = jnp.where(gt[0], ev, ti[0])
        for k in range(_TOPK):
            oidx[k, sl] = ti[k]
            oval[k, sl] = jnp.exp(t[k] - m) / s
        return carry

    lax.fori_loop(0, _ROWS // 16, group_body, 0)
    pltpu.sync_copy(oidx, idx_hbm.at[wid])
    pltpu.sync_copy(oval, val_hbm.at[wid])


@jax.jit
def kernel(hidden_states, weight):
    h = hidden_states.shape[-1]
    x = hidden_states.reshape(-1, h).astype(jnp.float32)
    logits_b = _tc_logits(x, weight.astype(jnp.float32))
    idx_b, val_b = _sc_topk(logits_b)
    idx = idx_b.transpose(0, 2, 1).reshape(_N, _TOPK)
    val = val_b.transpose(0, 2, 1).reshape(_N, _TOPK)
    return idx, val


# speedup vs baseline: 1.8311x; 1.8311x over previous
import functools

import jax
import jax.numpy as jnp
from jax.experimental import pallas as pl
from jax.experimental.pallas import tpu as pltpu

_TOPK = 8
_NE = 64
_BLK = 2048


def _gate_block(x_ref, w_ref, idx_ref, val_ref):
    x = x_ref[...]
    w = w_ref[...]
    logits = jax.lax.dot_general(
        w, x, (((1,), (1,)), ((), ())), preferred_element_type=jnp.float32
    )
    m = jnp.max(logits, axis=0, keepdims=True)
    e = jnp.exp(logits - m)
    s = jnp.sum(e, axis=0, keepdims=True)
    row = jax.lax.broadcasted_iota(jnp.int32, e.shape, 0).astype(jnp.float32)
    vals = e
    for k in range(_TOPK):
        mx = jnp.max(vals, axis=0, keepdims=True)
        idx = jnp.min(jnp.where(vals == mx, row, float(_NE)), axis=0, keepdims=True)
        idx_ref[k : k + 1, :] = idx.astype(jnp.int32)
        val_ref[k : k + 1, :] = mx / s
        vals = jnp.where(row == idx, -1.0, vals)


@jax.jit
def kernel(hidden_states, weight):
    h = hidden_states.shape[-1]
    x = hidden_states.reshape(-1, h).astype(jnp.float32)
    n = x.shape[0]
    grid = n // _BLK
    idx_t, val_t = pl.pallas_call(
        _gate_block,
        grid=(grid,),
        in_specs=[
            pl.BlockSpec((_BLK, h), lambda i: (i, 0)),
            pl.BlockSpec((_NE, h), lambda i: (0, 0)),
        ],
        out_specs=[
            pl.BlockSpec((_TOPK, _BLK), lambda i: (0, i)),
            pl.BlockSpec((_TOPK, _BLK), lambda i: (0, i)),
        ],
        out_shape=[
            jax.ShapeDtypeStruct((_TOPK, n), jnp.int32),
            jax.ShapeDtypeStruct((_TOPK, n), jnp.float32),
        ],
    )(x, weight.astype(jnp.float32))
    return idx_t.T, val_t.T


# speedup vs baseline: 1.8328x; 1.0009x over previous
import jax
import jax.numpy as jnp
from jax.experimental import pallas as pl

_TOPK = 8
_NE = 64
_BLK = 2048


def _gate_block(x_ref, w_ref, idx_ref, val_ref):
    x = x_ref[...]
    w = w_ref[...]
    logits = jax.lax.dot_general(
        w, x, (((1,), (1,)), ((), ())), preferred_element_type=jnp.float32
    )
    m = jnp.max(logits, axis=0, keepdims=True)
    e = jnp.exp(logits - m)
    s = jnp.sum(e, axis=0, keepdims=True)
    row = jax.lax.broadcasted_iota(jnp.int32, e.shape, 0).astype(jnp.float32)
    vals = e
    for k in range(_TOPK):
        mx = jnp.max(vals, axis=0, keepdims=True)
        idx = jnp.min(jnp.where(vals == mx, row, float(_NE)), axis=0, keepdims=True)
        idx_ref[k : k + 1, :] = idx.astype(jnp.int32)
        val_ref[k : k + 1, :] = mx / s
        vals = jnp.where(row == idx, -1.0, vals)


@jax.jit
def kernel(hidden_states, weight):
    h = hidden_states.shape[-1]
    x = hidden_states.reshape(-1, h).astype(jnp.float32)
    n = x.shape[0]
    grid = n // _BLK
    idx_t, val_t = pl.pallas_call(
        _gate_block,
        grid=(grid,),
        in_specs=[
            pl.BlockSpec((_BLK, h), lambda i: (i, 0)),
            pl.BlockSpec((_NE, h), lambda i: (0, 0)),
        ],
        out_specs=[
            pl.BlockSpec((_TOPK, _BLK), lambda i: (0, i)),
            pl.BlockSpec((_TOPK, _BLK), lambda i: (0, i)),
        ],
        out_shape=[
            jax.ShapeDtypeStruct((_TOPK, n), jnp.int32),
            jax.ShapeDtypeStruct((_TOPK, n), jnp.float32),
        ],
    )(x, weight.astype(jnp.float32))
    return idx_t.T, val_t.T
